# baseline (speedup 1.0000x reference)
import functools

import jax
import jax.numpy as jnp
from jax import lax
from jax.experimental import pallas as pl
from jax.experimental.pallas import tpu as pltpu
from jax.experimental.pallas import tpu_sc as plsc

N = 10000
E = 320000
B = 256

NUM_TILES = 16
NUM_CORES = 2
K = 128
NPAD = 10240
EPAD = 327680
ROW_BLK = 1000
NUM_BLKS = N // ROW_BLK

_F32 = jnp.float32
_HI = jax.lax.Precision.HIGHEST


def _dot(a, b):
    return jax.lax.dot_general(a, b, (((1,), (0,)), ((), ())),
                               precision=_HI, preferred_element_type=_F32)


def _sc_zero_acc(acc, rows_v, sid, ch):
    rows_per_tile = NPAD // NUM_TILES

    @pl.loop(0, K)
    def _(r):
        @pl.loop(0, ch // 16)
        def _(c):
            rows_v[r, pl.ds(c * 16, 16)] = jnp.zeros((16,), _F32)

    @pl.loop(0, rows_per_tile // K)
    def _(j):
        pltpu.sync_copy(rows_v, acc.at[pl.ds(sid * rows_per_tile + j * K, K)])


G = 16


def _sc_edge_pipeline(h_ref, srcE3, dstE3, w, src2d, dst2d, acc,
                      rows0, rows1, sem0, sem1, ssem0, ssem1, n_chunks):
    def wait_gather(buf, sem):
        pltpu.make_async_copy(h_ref.at[src2d.at[0]], buf, sem).wait()

    def wait_scatter(buf, sem):
        pltpu.make_async_copy(buf, acc.at[dst2d.at[0]], sem).wait()

    @pl.loop(0, n_chunks // G)
    def _(g):
        pltpu.sync_copy(srcE3.at[w, pl.ds(g * G, G)], src2d)
        pltpu.sync_copy(dstE3.at[w, pl.ds(g * G, G)], dst2d)
        pltpu.async_copy(h_ref.at[src2d.at[0]], rows0, sem0)
        pltpu.async_copy(h_ref.at[src2d.at[1]], rows1, sem1)

        @pl.loop(0, G // 2)
        def _(j):
            ci = 2 * j
            wait_gather(rows0, sem0)
            pltpu.async_copy(rows0, acc.at[dst2d.at[ci]], ssem0, add=True)
            wait_gather(rows1, sem1)
            pltpu.async_copy(rows1, acc.at[dst2d.at[ci + 1]], ssem1, add=True)
            n2 = jnp.minimum(ci + 2, G - 1)
            n3 = jnp.minimum(ci + 3, G - 1)
            wait_scatter(rows0, ssem0)
            pltpu.async_copy(h_ref.at[src2d.at[n2]], rows0, sem0)
            wait_scatter(rows1, ssem1)
            pltpu.async_copy(h_ref.at[src2d.at[n3]], rows1, sem1)

        wait_gather(rows0, sem0)
        wait_gather(rows1, sem1)


def _sc_writeout(acc, out_ref, sid):
    out_rows = 624
    tail = N - NUM_TILES * out_rows
    pltpu.sync_copy(acc.at[pl.ds(sid * out_rows, out_rows)],
                    out_ref.at[pl.ds(sid * out_rows, out_rows)])

    @pl.when(sid == 0)
    def _():
        pltpu.sync_copy(acc.at[pl.ds(NUM_TILES * out_rows, tail)],
                        out_ref.at[pl.ds(NUM_TILES * out_rows, tail)])


def _sc_scratch(ch, n_chunks):
    del n_chunks
    return [
        pltpu.VMEM_SHARED((NPAD, ch), _F32),
        pltpu.VMEM((G, K), jnp.int32),
        pltpu.VMEM((G, K), jnp.int32),
        pltpu.VMEM((K, ch), _F32),
        pltpu.VMEM((K, ch), _F32),
        pltpu.SemaphoreType.DMA,
        pltpu.SemaphoreType.DMA,
        pltpu.SemaphoreType.DMA,
        pltpu.SemaphoreType.DMA,
    ]


@functools.cache
def _make_sc_agg(ch):
    mesh = plsc.VectorSubcoreMesh(core_axis_name="c", subcore_axis_name="s")
    n_chunks = EPAD // (NUM_TILES * K)

    @functools.partial(
        pl.kernel,
        mesh=mesh,
        out_type=[jax.ShapeDtypeStruct((N, ch), _F32),
                  jax.ShapeDtypeStruct((N, ch), _F32)],
        scratch_types=_sc_scratch(ch, n_chunks),
    )
    def agg(hL, hR, srcE3, dstE3, aggL, aggR,
            acc, src2d, dst2d, rows0, rows1, sem0, sem1, ssem0, ssem1):
        cid = lax.axis_index("c")
        sid = lax.axis_index("s")
        _sc_zero_acc(acc, rows0, sid, ch)
        plsc.subcore_barrier()

        @pl.when(cid == 0)
        def _():
            _sc_edge_pipeline(hL, srcE3, dstE3, sid, src2d, dst2d, acc,
                              rows0, rows1, sem0, sem1, ssem0, ssem1,
                              n_chunks)

        @pl.when(cid == 1)
        def _():
            _sc_edge_pipeline(hR, srcE3, dstE3, sid, src2d, dst2d, acc,
                              rows0, rows1, sem0, sem1, ssem0, ssem1,
                              n_chunks)

        plsc.subcore_barrier()

        @pl.when(cid == 0)
        def _():
            _sc_writeout(acc, aggL, sid)

        @pl.when(cid == 1)
        def _():
            _sc_writeout(acc, aggR, sid)

    return agg


@functools.cache
def _make_sc_agg_full(ch):
    mesh = plsc.VectorSubcoreMesh(core_axis_name="c", subcore_axis_name="s")
    n_chunks = EPAD // (NUM_CORES * NUM_TILES * K)

    @functools.partial(
        pl.kernel,
        mesh=mesh,
        out_type=[jax.ShapeDtypeStruct((N, ch), _F32),
                  jax.ShapeDtypeStruct((N, ch), _F32)],
        scratch_types=_sc_scratch(ch, n_chunks),
    )
    def agg(h, srcE3, dstE3, agg_a, agg_b,
            acc, src2d, dst2d, rows0, rows1, sem0, sem1, ssem0, ssem1):
        cid = lax.axis_index("c")
        sid = lax.axis_index("s")
        _sc_zero_acc(acc, rows0, sid, ch)
        plsc.subcore_barrier()
        _sc_edge_pipeline(h, srcE3, dstE3, cid * NUM_TILES + sid,
                          src2d, dst2d, acc, rows0, rows1, sem0, sem1,
                          ssem0, ssem1, n_chunks)
        plsc.subcore_barrier()

        @pl.when(cid == 0)
        def _():
            _sc_writeout(acc, agg_a, sid)

        @pl.when(cid == 1)
        def _():
            _sc_writeout(acc, agg_b, sid)

    return agg


def _layer_body(split_agg, hL, hR, aL, aR, eps, w1, b1, g1, be1, w2, b2, g2,
                be2, outL, outR, z1s, z2s, s1, ss1, s2, ss2):
    p = pl.program_id(0)
    b = pl.program_id(1)

    @pl.when((p == 0) & (b == 0))
    def _():
        s1[...] = jnp.zeros_like(s1)
        ss1[...] = jnp.zeros_like(ss1)
        s2[...] = jnp.zeros_like(s2)
        ss2[...] = jnp.zeros_like(ss2)

    @pl.when(p == 0)
    def _():
        h = jnp.concatenate([hL[...], hR[...]], axis=1)
        if split_agg:
            a = jnp.concatenate([aL[...], aR[...]], axis=1)
        else:
            a = aL[...] + aR[...]
        y = (1.0 + eps[0, 0]) * h + a
        z1 = _dot(y, w1[...]) + b1[...]
        z1s[pl.ds(b * ROW_BLK, ROW_BLK), :] = z1
        s1[...] += jnp.sum(z1, axis=0, keepdims=True)
        ss1[...] += jnp.sum(z1 * z1, axis=0, keepdims=True)

    @pl.when(p == 1)
    def _():
        z1 = z1s[pl.ds(b * ROW_BLK, ROW_BLK), :]
        m = s1[...] / N
        v = ss1[...] / N - m * m
        a1 = (z1 - m) * jax.lax.rsqrt(v + 1e-5) * g1[...] + be1[...]
        a1 = jnp.maximum(a1, 0.0)
        z2 = _dot(a1, w2[...]) + b2[...]
        z2s[pl.ds(b * ROW_BLK, ROW_BLK), :] = z2
        s2[...] += jnp.sum(z2, axis=0, keepdims=True)
        ss2[...] += jnp.sum(z2 * z2, axis=0, keepdims=True)

    @pl.when(p == 2)
    def _():
        z2 = z2s[pl.ds(b * ROW_BLK, ROW_BLK), :]
        m = s2[...] / N
        v = ss2[...] / N - m * m
        hn = (z2 - m) * jax.lax.rsqrt(v + 1e-5) * g2[...] + be2[...]
        hn = jnp.maximum(hn, 0.0)
        half = hn.shape[1] // 2
        outL[...] = hn[:, :half]
        outR[...] = hn[:, half:]


@functools.cache
def _make_tc_layer(din, split_agg):
    chin = din // 2
    cha = chin if split_agg else din
    hid2 = 512
    hid = 256
    cho = hid // 2

    blk = lambda r, c: pl.BlockSpec(
        (r, c), lambda p, b: (jnp.where(p == 0, b, 0), 0))
    oblk = lambda r, c: pl.BlockSpec(
        (r, c), lambda p, b: (jnp.where(p == 2, b, 0), 0))
    full = lambda r, c: pl.BlockSpec((r, c), lambda p, b: (0, 0))

    return pl.pallas_call(
        functools.partial(_layer_body, split_agg),
        grid=(3, NUM_BLKS),
        in_specs=[
            blk(ROW_BLK, chin), blk(ROW_BLK, chin),
            blk(ROW_BLK, cha), blk(ROW_BLK, cha),
            full(1, 1),
            full(din, hid2), full(1, hid2),
            full(1, hid2), full(1, hid2),
            full(hid2, hid), full(1, hid),
            full(1, hid), full(1, hid),
        ],
        out_specs=[oblk(ROW_BLK, cho), oblk(ROW_BLK, cho)],
        out_shape=[jax.ShapeDtypeStruct((N, cho), _F32),
                   jax.ShapeDtypeStruct((N, cho), _F32)],
        scratch_shapes=[
            pltpu.VMEM((N, hid2), _F32),
            pltpu.VMEM((N, hid), _F32),
            pltpu.VMEM((1, hid2), _F32),
            pltpu.VMEM((1, hid2), _F32),
            pltpu.VMEM((1, hid), _F32),
            pltpu.VMEM((1, hid), _F32),
        ],
    )


def _head_body(hL, hR, batch, wc1, bc1, wc2, bc2, out):
    h = jnp.concatenate([hL[...], hR[...]], axis=1)
    onehot = (batch[...] == jax.lax.broadcasted_iota(jnp.int32, (N, B), 1))
    onehot = onehot.astype(_F32)
    sums = jax.lax.dot_general(onehot, h, (((0,), (0,)), ((), ())),
                               precision=_HI, preferred_element_type=_F32)
    counts = jax.lax.dot_general(onehot, jnp.ones((N, 1), _F32),
                                 (((0,), (0,)), ((), ())),
                                 precision=_HI, preferred_element_type=_F32)
    pooled = sums / jnp.maximum(counts, 1.0)
    hid = jnp.maximum(_dot(pooled, wc1[...]) + bc1[...], 0.0)
    out[...] = _dot(hid, wc2[...]) + bc2[...]


def _head(hL, hR, batch2d, wc1, bc1, wc2, bc2):
    nt = wc2.shape[1]
    return pl.pallas_call(
        _head_body,
        in_specs=[pl.BlockSpec(x.shape, lambda: (0,) * x.ndim)
                  for x in (hL, hR, batch2d, wc1, bc1, wc2, bc2)],
        out_specs=pl.BlockSpec((B, nt), lambda: (0, 0)),
        out_shape=jax.ShapeDtypeStruct((B, nt), _F32),
    )(hL, hR, batch2d, wc1, bc1, wc2, bc2)


def kernel(x, edge_index, batch, params):
    src = edge_index[0]
    dst = edge_index[1]
    npad = EPAD - E
    pad_src = (jnp.arange(npad, dtype=jnp.int32) * 37) % N
    pad_dst = N + (jnp.arange(npad, dtype=jnp.int32) % (NPAD - N))
    srcp = jnp.concatenate([src, pad_src])
    dstp = jnp.concatenate([dst, pad_dst])

    half = x.shape[1] // 2
    hL, hR = x[:, :half], x[:, half:]

    src_a = srcp.reshape(NUM_CORES * NUM_TILES, -1, K)
    dst_a = dstp.reshape(NUM_CORES * NUM_TILES, -1, K)
    src_b = srcp.reshape(NUM_TILES, -1, K)
    dst_b = dstp.reshape(NUM_TILES, -1, K)

    r1 = lambda a: a.reshape(1, -1)
    for i, lp in enumerate(params["layers"]):
        din = hL.shape[1] * 2
        if i == 0:
            aggA, aggB = _make_sc_agg_full(din)(x, src_a, dst_a)
        else:
            aggA, aggB = _make_sc_agg(hL.shape[1])(hL, hR, src_b, dst_b)
        hL, hR = _make_tc_layer(din, i != 0)(
            hL, hR, aggA, aggB, lp["eps"].reshape(1, 1),
            lp["W1"], r1(lp["b1"]), r1(lp["g1"]), r1(lp["be1"]),
            lp["W2"], r1(lp["b2"]), r1(lp["g2"]), r1(lp["be2"]))

    return _head(hL, hR, batch.reshape(N, 1),
                 params["Wc1"], params["bc1"].reshape(1, -1),
                 params["Wc2"], params["bc2"].reshape(1, -1))


# speedup vs baseline: 1.2860x; 1.2860x over previous
import functools

import jax
import jax.numpy as jnp
from jax import lax
from jax.experimental import pallas as pl
from jax.experimental.pallas import tpu as pltpu
from jax.experimental.pallas import tpu_sc as plsc

N = 10000
E = 320000
B = 256

NUM_TILES = 16
NUM_CORES = 2
K = 128
NPAD = 10240
EPAD = 327680
ROW_BLK = 1000
NUM_BLKS = N // ROW_BLK

_F32 = jnp.float32
_HI = jax.lax.Precision.DEFAULT


def _dot(a, b):
    return jax.lax.dot_general(a, b, (((1,), (0,)), ((), ())),
                               precision=_HI, preferred_element_type=_F32)


def _sc_zero_acc(acc, rows_v, sid, ch):
    rows_per_tile = NPAD // NUM_TILES

    @pl.loop(0, K)
    def _(r):
        @pl.loop(0, ch // 16)
        def _(c):
            rows_v[r, pl.ds(c * 16, 16)] = jnp.zeros((16,), _F32)

    @pl.loop(0, rows_per_tile // K)
    def _(j):
        pltpu.sync_copy(rows_v, acc.at[pl.ds(sid * rows_per_tile + j * K, K)])


def _sc_edge_pipeline(h_ref, srcE3, dstE3, w, src2d, dst2d, acc,
                      rows0, rows1, sem0, sem1, n_chunks, g_sz):
    @pl.loop(0, n_chunks // g_sz)
    def _(g):
        pltpu.sync_copy(srcE3.at[w, pl.ds(g * g_sz, g_sz)], src2d)
        pltpu.sync_copy(dstE3.at[w, pl.ds(g * g_sz, g_sz)], dst2d)
        pltpu.async_copy(h_ref.at[src2d.at[0]], rows0, sem0)

        @pl.loop(0, g_sz // 2)
        def _(j):
            ci = 2 * j
            pltpu.make_async_copy(h_ref.at[src2d.at[0]], rows0, sem0).wait()
            pltpu.async_copy(h_ref.at[src2d.at[ci + 1]], rows1, sem1)
            pltpu.sync_copy(rows0, acc.at[dst2d.at[ci]], add=True)
            nxt = jnp.minimum(ci + 2, g_sz - 1)
            pltpu.make_async_copy(h_ref.at[src2d.at[0]], rows1, sem1).wait()
            pltpu.async_copy(h_ref.at[src2d.at[nxt]], rows0, sem0)
            pltpu.sync_copy(rows1, acc.at[dst2d.at[ci + 1]], add=True)

        pltpu.make_async_copy(h_ref.at[src2d.at[0]], rows0, sem0).wait()


def _sc_writeout(acc, out_ref, sid):
    out_rows = 624
    tail = N - NUM_TILES * out_rows
    pltpu.sync_copy(acc.at[pl.ds(sid * out_rows, out_rows)],
                    out_ref.at[pl.ds(sid * out_rows, out_rows)])

    @pl.when(sid == 0)
    def _():
        pltpu.sync_copy(acc.at[pl.ds(NUM_TILES * out_rows, tail)],
                        out_ref.at[pl.ds(NUM_TILES * out_rows, tail)])


def _sc_scratch(ch, g_sz):
    return [
        pltpu.VMEM_SHARED((NPAD, ch), _F32),
        pltpu.VMEM((g_sz, K), jnp.int32),
        pltpu.VMEM((g_sz, K), jnp.int32),
        pltpu.VMEM((K, ch), _F32),
        pltpu.VMEM((K, ch), _F32),
        pltpu.SemaphoreType.DMA,
        pltpu.SemaphoreType.DMA,
    ]


@functools.cache
def _make_sc_agg(ch):
    mesh = plsc.VectorSubcoreMesh(core_axis_name="c", subcore_axis_name="s")
    n_chunks = EPAD // (NUM_TILES * K)
    g_sz = 32

    @functools.partial(
        pl.kernel,
        mesh=mesh,
        out_type=[jax.ShapeDtypeStruct((N, ch), _F32),
                  jax.ShapeDtypeStruct((N, ch), _F32)],
        scratch_types=_sc_scratch(ch, g_sz),
    )
    def agg(hL, hR, srcE3, dstE3, aggL, aggR,
            acc, src2d, dst2d, rows0, rows1, sem0, sem1):
        cid = lax.axis_index("c")
        sid = lax.axis_index("s")
        _sc_zero_acc(acc, rows0, sid, ch)
        plsc.subcore_barrier()

        @pl.when(cid == 0)
        def _():
            _sc_edge_pipeline(hL, srcE3, dstE3, sid, src2d, dst2d, acc,
                              rows0, rows1, sem0, sem1, n_chunks, g_sz)

        @pl.when(cid == 1)
        def _():
            _sc_edge_pipeline(hR, srcE3, dstE3, sid, src2d, dst2d, acc,
                              rows0, rows1, sem0, sem1, n_chunks, g_sz)

        plsc.subcore_barrier()

        @pl.when(cid == 0)
        def _():
            _sc_writeout(acc, aggL, sid)

        @pl.when(cid == 1)
        def _():
            _sc_writeout(acc, aggR, sid)

    return agg


@functools.cache
def _make_sc_agg_full(ch):
    mesh = plsc.VectorSubcoreMesh(core_axis_name="c", subcore_axis_name="s")
    n_chunks = EPAD // (NUM_CORES * NUM_TILES * K)
    g_sz = 16

    @functools.partial(
        pl.kernel,
        mesh=mesh,
        out_type=[jax.ShapeDtypeStruct((N, ch), _F32),
                  jax.ShapeDtypeStruct((N, ch), _F32)],
        scratch_types=_sc_scratch(ch, g_sz),
    )
    def agg(h, srcE3, dstE3, agg_a, agg_b,
            acc, src2d, dst2d, rows0, rows1, sem0, sem1):
        cid = lax.axis_index("c")
        sid = lax.axis_index("s")
        _sc_zero_acc(acc, rows0, sid, ch)
        plsc.subcore_barrier()
        _sc_edge_pipeline(h, srcE3, dstE3, cid * NUM_TILES + sid,
                          src2d, dst2d, acc, rows0, rows1, sem0, sem1,
                          n_chunks, g_sz)
        plsc.subcore_barrier()

        @pl.when(cid == 0)
        def _():
            _sc_writeout(acc, agg_a, sid)

        @pl.when(cid == 1)
        def _():
            _sc_writeout(acc, agg_b, sid)

    return agg


def _layer_body(split_agg, hL, hR, aL, aR, eps, w1, b1, g1, be1, w2, b2, g2,
                be2, outL, outR, z1s, z2s, s1, ss1, s2, ss2):
    p = pl.program_id(0)
    b = pl.program_id(1)

    @pl.when((p == 0) & (b == 0))
    def _():
        s1[...] = jnp.zeros_like(s1)
        ss1[...] = jnp.zeros_like(ss1)
        s2[...] = jnp.zeros_like(s2)
        ss2[...] = jnp.zeros_like(ss2)

    @pl.when(p == 0)
    def _():
        h = jnp.concatenate([hL[...], hR[...]], axis=1)
        if split_agg:
            a = jnp.concatenate([aL[...], aR[...]], axis=1)
        else:
            a = aL[...] + aR[...]
        y = (1.0 + eps[0, 0]) * h + a
        z1 = _dot(y, w1[...]) + b1[...]
        z1s[pl.ds(b * ROW_BLK, ROW_BLK), :] = z1
        s1[...] += jnp.sum(z1, axis=0, keepdims=True)
        ss1[...] += jnp.sum(z1 * z1, axis=0, keepdims=True)

    @pl.when(p == 1)
    def _():
        z1 = z1s[pl.ds(b * ROW_BLK, ROW_BLK), :]
        m = s1[...] / N
        v = ss1[...] / N - m * m
        a1 = (z1 - m) * jax.lax.rsqrt(v + 1e-5) * g1[...] + be1[...]
        a1 = jnp.maximum(a1, 0.0)
        z2 = _dot(a1, w2[...]) + b2[...]
        z2s[pl.ds(b * ROW_BLK, ROW_BLK), :] = z2
        s2[...] += jnp.sum(z2, axis=0, keepdims=True)
        ss2[...] += jnp.sum(z2 * z2, axis=0, keepdims=True)

    @pl.when(p == 2)
    def _():
        z2 = z2s[pl.ds(b * ROW_BLK, ROW_BLK), :]
        m = s2[...] / N
        v = ss2[...] / N - m * m
        hn = (z2 - m) * jax.lax.rsqrt(v + 1e-5) * g2[...] + be2[...]
        hn = jnp.maximum(hn, 0.0)
        half = hn.shape[1] // 2
        outL[...] = hn[:, :half]
        outR[...] = hn[:, half:]


@functools.cache
def _make_tc_layer(din, split_agg):
    chin = din // 2
    cha = chin if split_agg else din
    hid2 = 512
    hid = 256
    cho = hid // 2

    blk = lambda r, c: pl.BlockSpec(
        (r, c), lambda p, b: (jnp.where(p == 0, b, 0), 0))
    oblk = lambda r, c: pl.BlockSpec(
        (r, c), lambda p, b: (jnp.where(p == 2, b, 0), 0))
    full = lambda r, c: pl.BlockSpec((r, c), lambda p, b: (0, 0))

    return pl.pallas_call(
        functools.partial(_layer_body, split_agg),
        grid=(3, NUM_BLKS),
        in_specs=[
            blk(ROW_BLK, chin), blk(ROW_BLK, chin),
            blk(ROW_BLK, cha), blk(ROW_BLK, cha),
            full(1, 1),
            full(din, hid2), full(1, hid2),
            full(1, hid2), full(1, hid2),
            full(hid2, hid), full(1, hid),
            full(1, hid), full(1, hid),
        ],
        out_specs=[oblk(ROW_BLK, cho), oblk(ROW_BLK, cho)],
        out_shape=[jax.ShapeDtypeStruct((N, cho), _F32),
                   jax.ShapeDtypeStruct((N, cho), _F32)],
        scratch_shapes=[
            pltpu.VMEM((N, hid2), _F32),
            pltpu.VMEM((N, hid), _F32),
            pltpu.VMEM((1, hid2), _F32),
            pltpu.VMEM((1, hid2), _F32),
            pltpu.VMEM((1, hid), _F32),
            pltpu.VMEM((1, hid), _F32),
        ],
    )


def _head_body(hL, hR, batch, wc1, bc1, wc2, bc2, out):
    h = jnp.concatenate([hL[...], hR[...]], axis=1)
    onehot = (batch[...] == jax.lax.broadcasted_iota(jnp.int32, (N, B), 1))
    onehot = onehot.astype(_F32)
    sums = jax.lax.dot_general(onehot, h, (((0,), (0,)), ((), ())),
                               precision=_HI, preferred_element_type=_F32)
    counts = jax.lax.dot_general(onehot, jnp.ones((N, 1), _F32),
                                 (((0,), (0,)), ((), ())),
                                 precision=_HI, preferred_element_type=_F32)
    pooled = sums / jnp.maximum(counts, 1.0)
    hid = jnp.maximum(_dot(pooled, wc1[...]) + bc1[...], 0.0)
    out[...] = _dot(hid, wc2[...]) + bc2[...]


def _head(hL, hR, batch2d, wc1, bc1, wc2, bc2):
    nt = wc2.shape[1]
    return pl.pallas_call(
        _head_body,
        in_specs=[pl.BlockSpec(x.shape, lambda: (0,) * x.ndim)
                  for x in (hL, hR, batch2d, wc1, bc1, wc2, bc2)],
        out_specs=pl.BlockSpec((B, nt), lambda: (0, 0)),
        out_shape=jax.ShapeDtypeStruct((B, nt), _F32),
    )(hL, hR, batch2d, wc1, bc1, wc2, bc2)


def kernel(x, edge_index, batch, params):
    src = edge_index[0]
    dst = edge_index[1]
    npad = EPAD - E
    pad_src = (jnp.arange(npad, dtype=jnp.int32) * 37) % N
    pad_dst = N + (jnp.arange(npad, dtype=jnp.int32) % (NPAD - N))
    srcp = jnp.concatenate([src, pad_src])
    dstp = jnp.concatenate([dst, pad_dst])

    half = x.shape[1] // 2
    hL, hR = x[:, :half], x[:, half:]

    src_a = srcp.reshape(NUM_CORES * NUM_TILES, -1, K)
    dst_a = dstp.reshape(NUM_CORES * NUM_TILES, -1, K)
    src_b = srcp.reshape(NUM_TILES, -1, K)
    dst_b = dstp.reshape(NUM_TILES, -1, K)

    r1 = lambda a: a.reshape(1, -1)
    for i, lp in enumerate(params["layers"]):
        din = hL.shape[1] * 2
        if i == 0:
            aggA, aggB = _make_sc_agg_full(din)(x, src_a, dst_a)
        else:
            aggA, aggB = _make_sc_agg(hL.shape[1])(hL, hR, src_b, dst_b)
        hL, hR = _make_tc_layer(din, i != 0)(
            hL, hR, aggA, aggB, lp["eps"].reshape(1, 1),
            lp["W1"], r1(lp["b1"]), r1(lp["g1"]), r1(lp["be1"]),
            lp["W2"], r1(lp["b2"]), r1(lp["g2"]), r1(lp["be2"]))

    return _head(hL, hR, batch.reshape(N, 1),
                 params["Wc1"], params["bc1"].reshape(1, -1),
                 params["Wc2"], params["bc2"].reshape(1, -1))


# speedup vs baseline: 1.2952x; 1.0071x over previous
import functools

import jax
import jax.numpy as jnp
from jax import lax
from jax.experimental import pallas as pl
from jax.experimental.pallas import tpu as pltpu
from jax.experimental.pallas import tpu_sc as plsc

N = 10000
E = 320000
B = 256

NUM_TILES = 16
NUM_CORES = 2
K = 128
NPAD = 10240
EPAD = 327680
ROW_BLK = 1000
NUM_BLKS = N // ROW_BLK

_F32 = jnp.float32
_HI = jax.lax.Precision.DEFAULT


def _dot(a, b):
    return jax.lax.dot_general(a, b, (((1,), (0,)), ((), ())),
                               precision=_HI, preferred_element_type=_F32)


def _sc_zero_acc(acc, rows_v, sid, ch):
    rows_per_tile = NPAD // NUM_TILES

    @pl.loop(0, K)
    def _(r):
        @pl.loop(0, ch // 16)
        def _(c):
            rows_v[r, pl.ds(c * 16, 16)] = jnp.zeros((16,), _F32)

    @pl.loop(0, rows_per_tile // K)
    def _(j):
        pltpu.sync_copy(rows_v, acc.at[pl.ds(sid * rows_per_tile + j * K, K)])


def _sc_edge_pipeline(h_ref, srcE3, dstE3, w, src2d, dst2d, acc,
                      rows0, rows1, sem0, sem1, n_chunks, g_sz):
    @pl.loop(0, n_chunks // g_sz)
    def _(g):
        pltpu.sync_copy(srcE3.at[w, pl.ds(g * g_sz, g_sz)], src2d)
        pltpu.sync_copy(dstE3.at[w, pl.ds(g * g_sz, g_sz)], dst2d)
        pltpu.async_copy(h_ref.at[src2d.at[0]], rows0, sem0)

        @pl.loop(0, g_sz // 2)
        def _(j):
            ci = 2 * j
            pltpu.make_async_copy(h_ref.at[src2d.at[0]], rows0, sem0).wait()
            pltpu.async_copy(h_ref.at[src2d.at[ci + 1]], rows1, sem1)
            pltpu.sync_copy(rows0, acc.at[dst2d.at[ci]], add=True)
            nxt = jnp.minimum(ci + 2, g_sz - 1)
            pltpu.make_async_copy(h_ref.at[src2d.at[0]], rows1, sem1).wait()
            pltpu.async_copy(h_ref.at[src2d.at[nxt]], rows0, sem0)
            pltpu.sync_copy(rows1, acc.at[dst2d.at[ci + 1]], add=True)

        pltpu.make_async_copy(h_ref.at[src2d.at[0]], rows0, sem0).wait()


def _sc_writeout(acc, out_ref, sid):
    out_rows = 624
    tail = N - NUM_TILES * out_rows
    pltpu.sync_copy(acc.at[pl.ds(sid * out_rows, out_rows)],
                    out_ref.at[pl.ds(sid * out_rows, out_rows)])

    @pl.when(sid == 0)
    def _():
        pltpu.sync_copy(acc.at[pl.ds(NUM_TILES * out_rows, tail)],
                        out_ref.at[pl.ds(NUM_TILES * out_rows, tail)])


def _sc_scratch(ch, g_sz):
    return [
        pltpu.VMEM_SHARED((NPAD, ch), _F32),
        pltpu.VMEM((g_sz, K), jnp.int32),
        pltpu.VMEM((g_sz, K), jnp.int32),
        pltpu.VMEM((K, ch), _F32),
        pltpu.VMEM((K, ch), _F32),
        pltpu.SemaphoreType.DMA,
        pltpu.SemaphoreType.DMA,
    ]


@functools.cache
def _make_sc_agg(ch):
    mesh = plsc.VectorSubcoreMesh(core_axis_name="c", subcore_axis_name="s")
    n_chunks = EPAD // (NUM_TILES * K)
    g_sz = 32

    @functools.partial(
        pl.kernel,
        mesh=mesh,
        out_type=[jax.ShapeDtypeStruct((N, ch), _F32),
                  jax.ShapeDtypeStruct((N, ch), _F32)],
        scratch_types=_sc_scratch(ch, g_sz),
    )
    def agg(hL, hR, srcE3, dstE3, aggL, aggR,
            acc, src2d, dst2d, rows0, rows1, sem0, sem1):
        cid = lax.axis_index("c")
        sid = lax.axis_index("s")
        _sc_zero_acc(acc, rows0, sid, ch)
        plsc.subcore_barrier()

        @pl.when(cid == 0)
        def _():
            _sc_edge_pipeline(hL, srcE3, dstE3, sid, src2d, dst2d, acc,
                              rows0, rows1, sem0, sem1, n_chunks, g_sz)

        @pl.when(cid == 1)
        def _():
            _sc_edge_pipeline(hR, srcE3, dstE3, sid, src2d, dst2d, acc,
                              rows0, rows1, sem0, sem1, n_chunks, g_sz)

        plsc.subcore_barrier()

        @pl.when(cid == 0)
        def _():
            _sc_writeout(acc, aggL, sid)

        @pl.when(cid == 1)
        def _():
            _sc_writeout(acc, aggR, sid)

    return agg


@functools.cache
def _make_sc_agg_full(ch):
    mesh = plsc.VectorSubcoreMesh(core_axis_name="c", subcore_axis_name="s")
    n_chunks = EPAD // (NUM_CORES * NUM_TILES * K)
    g_sz = 16

    @functools.partial(
        pl.kernel,
        mesh=mesh,
        out_type=[jax.ShapeDtypeStruct((N, ch), _F32),
                  jax.ShapeDtypeStruct((N, ch), _F32)],
        scratch_types=_sc_scratch(ch, g_sz),
    )
    def agg(h, srcE3, dstE3, agg_a, agg_b,
            acc, src2d, dst2d, rows0, rows1, sem0, sem1):
        cid = lax.axis_index("c")
        sid = lax.axis_index("s")
        _sc_zero_acc(acc, rows0, sid, ch)
        plsc.subcore_barrier()
        _sc_edge_pipeline(h, srcE3, dstE3, cid * NUM_TILES + sid,
                          src2d, dst2d, acc, rows0, rows1, sem0, sem1,
                          n_chunks, g_sz)
        plsc.subcore_barrier()

        @pl.when(cid == 0)
        def _():
            _sc_writeout(acc, agg_a, sid)

        @pl.when(cid == 1)
        def _():
            _sc_writeout(acc, agg_b, sid)

    return agg


def _layer_body(split_agg, hL, hR, aL, aR, eps, w1, b1, g1, be1, w2, b2, g2,
                be2, outL, outR, z1s, z2s, s1, ss1, s2, ss2):
    p = pl.program_id(0)
    b = pl.program_id(1)

    @pl.when((p == 0) & (b == 0))
    def _():
        s1[...] = jnp.zeros_like(s1)
        ss1[...] = jnp.zeros_like(ss1)
        s2[...] = jnp.zeros_like(s2)
        ss2[...] = jnp.zeros_like(ss2)

    @pl.when(p == 0)
    def _():
        h = jnp.concatenate([hL[...], hR[...]], axis=1)
        if split_agg:
            a = jnp.concatenate([aL[...], aR[...]], axis=1)
        else:
            a = aL[...] + aR[...]
        y = (1.0 + eps[0, 0]) * h + a
        z1 = _dot(y, w1[...]) + b1[...]
        z1s[pl.ds(b * ROW_BLK, ROW_BLK), :] = z1
        s1[...] += jnp.sum(z1, axis=0, keepdims=True)
        ss1[...] += jnp.sum(z1 * z1, axis=0, keepdims=True)

    @pl.when(p == 1)
    def _():
        z1 = z1s[pl.ds(b * ROW_BLK, ROW_BLK), :]
        m = s1[...] / N
        v = ss1[...] / N - m * m
        a1 = (z1 - m) * jax.lax.rsqrt(v + 1e-5) * g1[...] + be1[...]
        a1 = jnp.maximum(a1, 0.0)
        z2 = _dot(a1, w2[...]) + b2[...]
        z2s[pl.ds(b * ROW_BLK, ROW_BLK), :] = z2
        s2[...] += jnp.sum(z2, axis=0, keepdims=True)
        ss2[...] += jnp.sum(z2 * z2, axis=0, keepdims=True)

    @pl.when(p == 2)
    def _():
        z2 = z2s[pl.ds(b * ROW_BLK, ROW_BLK), :]
        m = s2[...] / N
        v = ss2[...] / N - m * m
        hn = (z2 - m) * jax.lax.rsqrt(v + 1e-5) * g2[...] + be2[...]
        hn = jnp.maximum(hn, 0.0)
        half = hn.shape[1] // 2
        outL[...] = hn[:, :half]
        outR[...] = hn[:, half:]


def _head_layer_body(split_agg, nt,
                     hL, hR, aL, aR, eps, w1, b1, g1, be1, w2, b2, g2, be2,
                     batch, wc1, bc1, wc2, bc2, out,
                     z1s, z2s, s1, ss1, s2, ss2, psum, pcnt):
    p = pl.program_id(0)
    b = pl.program_id(1)

    @pl.when((p == 0) & (b == 0))
    def _():
        for ref in (s1, ss1, s2, ss2, psum, pcnt):
            ref[...] = jnp.zeros_like(ref)

    @pl.when(p == 0)
    def _():
        h = jnp.concatenate([hL[...], hR[...]], axis=1)
        if split_agg:
            a = jnp.concatenate([aL[...], aR[...]], axis=1)
        else:
            a = aL[...] + aR[...]
        y = (1.0 + eps[0, 0]) * h + a
        z1 = _dot(y, w1[...]) + b1[...]
        z1s[pl.ds(b * ROW_BLK, ROW_BLK), :] = z1
        s1[...] += jnp.sum(z1, axis=0, keepdims=True)
        ss1[...] += jnp.sum(z1 * z1, axis=0, keepdims=True)

    @pl.when(p == 1)
    def _():
        z1 = z1s[pl.ds(b * ROW_BLK, ROW_BLK), :]
        m = s1[...] / N
        v = ss1[...] / N - m * m
        a1 = (z1 - m) * jax.lax.rsqrt(v + 1e-5) * g1[...] + be1[...]
        a1 = jnp.maximum(a1, 0.0)
        z2 = _dot(a1, w2[...]) + b2[...]
        z2s[pl.ds(b * ROW_BLK, ROW_BLK), :] = z2
        s2[...] += jnp.sum(z2, axis=0, keepdims=True)
        ss2[...] += jnp.sum(z2 * z2, axis=0, keepdims=True)

    @pl.when(p == 2)
    def _():
        z2 = z2s[pl.ds(b * ROW_BLK, ROW_BLK), :]
        m = s2[...] / N
        v = ss2[...] / N - m * m
        hn = (z2 - m) * jax.lax.rsqrt(v + 1e-5) * g2[...] + be2[...]
        hn = jnp.maximum(hn, 0.0)
        oh = (batch[...] ==
              jax.lax.broadcasted_iota(jnp.int32, (ROW_BLK, B), 1))
        oh = oh.astype(_F32)
        psum[...] += jax.lax.dot_general(
            oh, hn, (((0,), (0,)), ((), ())), precision=_HI,
            preferred_element_type=_F32)
        pcnt[...] += jax.lax.dot_general(
            oh, jnp.ones((ROW_BLK, 128), _F32), (((0,), (0,)), ((), ())),
            precision=_HI, preferred_element_type=_F32)

    @pl.when((p == 3) & (b == 0))
    def _():
        pooled = psum[...] / jnp.maximum(pcnt[...][:, :1], 1.0)
        hid = jnp.maximum(_dot(pooled, wc1[...]) + bc1[...], 0.0)
        out[...] = _dot(hid, wc2[...]) + bc2[...]


@functools.cache
def _make_tc_layer(din, split_agg, nt=0):
    chin = din // 2
    cha = chin if split_agg else din
    hid2 = 512
    hid = 256
    cho = hid // 2
    fuse_head = nt > 0

    blk = lambda r, c: pl.BlockSpec(
        (r, c), lambda p, b: (jnp.where(p == 0, b, 0), 0))
    p2blk = lambda r, c: pl.BlockSpec(
        (r, c), lambda p, b: (jnp.where(p == 2, b, 0), 0))
    full = lambda r, c: pl.BlockSpec((r, c), lambda p, b: (0, 0))

    in_specs = [
        blk(ROW_BLK, chin), blk(ROW_BLK, chin),
        blk(ROW_BLK, cha), blk(ROW_BLK, cha),
        full(1, 1),
        full(din, hid2), full(1, hid2),
        full(1, hid2), full(1, hid2),
        full(hid2, hid), full(1, hid),
        full(1, hid), full(1, hid),
    ]
    scratch = [
        pltpu.VMEM((N, hid2), _F32),
        pltpu.VMEM((N, hid), _F32),
        pltpu.VMEM((1, hid2), _F32),
        pltpu.VMEM((1, hid2), _F32),
        pltpu.VMEM((1, hid), _F32),
        pltpu.VMEM((1, hid), _F32),
    ]
    if fuse_head:
        in_specs += [
            p2blk(ROW_BLK, 1),
            full(hid, hid // 2), full(1, hid // 2),
            full(hid // 2, nt), full(1, nt),
        ]
        scratch += [pltpu.VMEM((B, hid), _F32), pltpu.VMEM((B, 128), _F32)]
        return pl.pallas_call(
            functools.partial(_head_layer_body, split_agg, nt),
            grid=(4, NUM_BLKS),
            in_specs=in_specs,
            out_specs=[full(B, nt)],
            out_shape=[jax.ShapeDtypeStruct((B, nt), _F32)],
            scratch_shapes=scratch,
        )
    return pl.pallas_call(
        functools.partial(_layer_body, split_agg),
        grid=(3, NUM_BLKS),
        in_specs=in_specs,
        out_specs=[p2blk(ROW_BLK, cho), p2blk(ROW_BLK, cho)],
        out_shape=[jax.ShapeDtypeStruct((N, cho), _F32),
                   jax.ShapeDtypeStruct((N, cho), _F32)],
        scratch_shapes=scratch,
    )


def kernel(x, edge_index, batch, params):
    src = edge_index[0]
    dst = edge_index[1]
    npad = EPAD - E
    pad_src = (jnp.arange(npad, dtype=jnp.int32) * 37) % N
    pad_dst = N + (jnp.arange(npad, dtype=jnp.int32) % (NPAD - N))
    srcp = jnp.concatenate([src, pad_src])
    dstp = jnp.concatenate([dst, pad_dst])

    half = x.shape[1] // 2
    hL, hR = x[:, :half], x[:, half:]

    src_a = srcp.reshape(NUM_CORES * NUM_TILES, -1, K)
    dst_a = dstp.reshape(NUM_CORES * NUM_TILES, -1, K)
    src_b = srcp.reshape(NUM_TILES, -1, K)
    dst_b = dstp.reshape(NUM_TILES, -1, K)

    r1 = lambda a: a.reshape(1, -1)
    nt = params["Wc2"].shape[1]
    for i, lp in enumerate(params["layers"]):
        din = hL.shape[1] * 2
        if i == 0:
            aggA, aggB = _make_sc_agg_full(din)(x, src_a, dst_a)
        else:
            aggA, aggB = _make_sc_agg(hL.shape[1])(hL, hR, src_b, dst_b)
        args = [hL, hR, aggA, aggB, lp["eps"].reshape(1, 1),
                lp["W1"], r1(lp["b1"]), r1(lp["g1"]), r1(lp["be1"]),
                lp["W2"], r1(lp["b2"]), r1(lp["g2"]), r1(lp["be2"])]
        last = i == len(params["layers"]) - 1
        if last:
            args += [batch.reshape(N, 1),
                     params["Wc1"], r1(params["bc1"]),
                     params["Wc2"], r1(params["bc2"])]
            (logits,) = _make_tc_layer(din, i != 0, nt)(*args)
            return logits
        hL, hR = _make_tc_layer(din, i != 0)(*args)


# speedup vs baseline: 1.3066x; 1.0088x over previous
import functools

import jax
import jax.numpy as jnp
from jax import lax
from jax.experimental import pallas as pl
from jax.experimental.pallas import tpu as pltpu
from jax.experimental.pallas import tpu_sc as plsc

N = 10000
E = 320000
B = 256

NUM_TILES = 16
NUM_CORES = 2
K = 128
NPAD = 10240
EPAD = 327680
ROW_BLK = 1000
NUM_BLKS = N // ROW_BLK

_F32 = jnp.float32
_HI = jax.lax.Precision.DEFAULT


def _dot(a, b):
    return jax.lax.dot_general(a, b, (((1,), (0,)), ((), ())),
                               precision=_HI, preferred_element_type=_F32)


def _sc_zero_acc(acc, rows_v, sid, ch):
    rows_per_tile = NPAD // NUM_TILES

    @pl.loop(0, K)
    def _(r):
        @pl.loop(0, ch // 16)
        def _(c):
            rows_v[r, pl.ds(c * 16, 16)] = jnp.zeros((16,), _F32)

    @pl.loop(0, rows_per_tile // K)
    def _(j):
        pltpu.sync_copy(rows_v, acc.at[pl.ds(sid * rows_per_tile + j * K, K)])


def _sc_edge_pipeline(h_ref, srcE3, dstE3, w, src2d, dst2d, acc,
                      rows0, rows1, sem0, sem1, n_chunks, g_sz):
    @pl.loop(0, n_chunks // g_sz)
    def _(g):
        pltpu.async_copy(srcE3.at[w, pl.ds(g * g_sz, g_sz)], src2d, sem0)
        pltpu.async_copy(dstE3.at[w, pl.ds(g * g_sz, g_sz)], dst2d, sem1)
        pltpu.make_async_copy(srcE3.at[w, pl.ds(g * g_sz, g_sz)],
                              src2d, sem0).wait()
        pltpu.make_async_copy(dstE3.at[w, pl.ds(g * g_sz, g_sz)],
                              dst2d, sem1).wait()
        pltpu.async_copy(h_ref.at[src2d.at[0]], rows0, sem0)

        @pl.loop(0, g_sz // 2)
        def _(j):
            ci = 2 * j
            pltpu.make_async_copy(h_ref.at[src2d.at[0]], rows0, sem0).wait()
            pltpu.async_copy(h_ref.at[src2d.at[ci + 1]], rows1, sem1)
            pltpu.sync_copy(rows0, acc.at[dst2d.at[ci]], add=True)
            nxt = jnp.minimum(ci + 2, g_sz - 1)
            pltpu.make_async_copy(h_ref.at[src2d.at[0]], rows1, sem1).wait()
            pltpu.async_copy(h_ref.at[src2d.at[nxt]], rows0, sem0)
            pltpu.sync_copy(rows1, acc.at[dst2d.at[ci + 1]], add=True)

        pltpu.make_async_copy(h_ref.at[src2d.at[0]], rows0, sem0).wait()


def _sc_writeout(acc, out_ref, sid):
    out_rows = 624
    tail = N - NUM_TILES * out_rows
    pltpu.sync_copy(acc.at[pl.ds(sid * out_rows, out_rows)],
                    out_ref.at[pl.ds(sid * out_rows, out_rows)])

    @pl.when(sid == 0)
    def _():
        pltpu.sync_copy(acc.at[pl.ds(NUM_TILES * out_rows, tail)],
                        out_ref.at[pl.ds(NUM_TILES * out_rows, tail)])


def _sc_scratch(ch, g_sz):
    return [
        pltpu.VMEM_SHARED((NPAD, ch), _F32),
        pltpu.VMEM((g_sz, K), jnp.int32),
        pltpu.VMEM((g_sz, K), jnp.int32),
        pltpu.VMEM((K, ch), _F32),
        pltpu.VMEM((K, ch), _F32),
        pltpu.SemaphoreType.DMA,
        pltpu.SemaphoreType.DMA,
    ]


@functools.cache
def _make_sc_agg(ch):
    mesh = plsc.VectorSubcoreMesh(core_axis_name="c", subcore_axis_name="s")
    n_chunks = EPAD // (NUM_TILES * K)
    g_sz = 32

    @functools.partial(
        pl.kernel,
        mesh=mesh,
        out_type=[jax.ShapeDtypeStruct((N, ch), _F32),
                  jax.ShapeDtypeStruct((N, ch), _F32)],
        scratch_types=_sc_scratch(ch, g_sz),
    )
    def agg(hL, hR, srcE3, dstE3, aggL, aggR,
            acc, src2d, dst2d, rows0, rows1, sem0, sem1):
        cid = lax.axis_index("c")
        sid = lax.axis_index("s")
        _sc_zero_acc(acc, rows0, sid, ch)
        plsc.subcore_barrier()

        @pl.when(cid == 0)
        def _():
            _sc_edge_pipeline(hL, srcE3, dstE3, sid, src2d, dst2d, acc,
                              rows0, rows1, sem0, sem1, n_chunks, g_sz)

        @pl.when(cid == 1)
        def _():
            _sc_edge_pipeline(hR, srcE3, dstE3, sid, src2d, dst2d, acc,
                              rows0, rows1, sem0, sem1, n_chunks, g_sz)

        plsc.subcore_barrier()

        @pl.when(cid == 0)
        def _():
            _sc_writeout(acc, aggL, sid)

        @pl.when(cid == 1)
        def _():
            _sc_writeout(acc, aggR, sid)

    return agg


@functools.cache
def _make_sc_agg_full(ch):
    mesh = plsc.VectorSubcoreMesh(core_axis_name="c", subcore_axis_name="s")
    n_chunks = EPAD // (NUM_CORES * NUM_TILES * K)
    g_sz = 16

    @functools.partial(
        pl.kernel,
        mesh=mesh,
        out_type=[jax.ShapeDtypeStruct((N, ch), _F32),
                  jax.ShapeDtypeStruct((N, ch), _F32)],
        scratch_types=_sc_scratch(ch, g_sz),
    )
    def agg(h, srcE3, dstE3, agg_a, agg_b,
            acc, src2d, dst2d, rows0, rows1, sem0, sem1):
        cid = lax.axis_index("c")
        sid = lax.axis_index("s")
        _sc_zero_acc(acc, rows0, sid, ch)
        plsc.subcore_barrier()
        _sc_edge_pipeline(h, srcE3, dstE3, cid * NUM_TILES + sid,
                          src2d, dst2d, acc, rows0, rows1, sem0, sem1,
                          n_chunks, g_sz)
        plsc.subcore_barrier()

        @pl.when(cid == 0)
        def _():
            _sc_writeout(acc, agg_a, sid)

        @pl.when(cid == 1)
        def _():
            _sc_writeout(acc, agg_b, sid)

    return agg


def _layer_body(split_agg, hL, hR, aL, aR, eps, w1, b1, g1, be1, w2, b2, g2,
                be2, outL, outR, z1s, z2s, s1, ss1, s2, ss2):
    p = pl.program_id(0)
    b = pl.program_id(1)

    @pl.when((p == 0) & (b == 0))
    def _():
        s1[...] = jnp.zeros_like(s1)
        ss1[...] = jnp.zeros_like(ss1)
        s2[...] = jnp.zeros_like(s2)
        ss2[...] = jnp.zeros_like(ss2)

    @pl.when(p == 0)
    def _():
        h = jnp.concatenate([hL[...], hR[...]], axis=1)
        if split_agg:
            a = jnp.concatenate([aL[...], aR[...]], axis=1)
        else:
            a = aL[...] + aR[...]
        y = (1.0 + eps[0, 0]) * h + a
        z1 = _dot(y, w1[...]) + b1[...]
        z1s[pl.ds(b * ROW_BLK, ROW_BLK), :] = z1
        s1[...] += jnp.sum(z1, axis=0, keepdims=True)
        ss1[...] += jnp.sum(z1 * z1, axis=0, keepdims=True)

    @pl.when(p == 1)
    def _():
        z1 = z1s[pl.ds(b * ROW_BLK, ROW_BLK), :]
        m = s1[...] / N
        v = ss1[...] / N - m * m
        a1 = (z1 - m) * jax.lax.rsqrt(v + 1e-5) * g1[...] + be1[...]
        a1 = jnp.maximum(a1, 0.0)
        z2 = _dot(a1, w2[...]) + b2[...]
        z2s[pl.ds(b * ROW_BLK, ROW_BLK), :] = z2
        s2[...] += jnp.sum(z2, axis=0, keepdims=True)
        ss2[...] += jnp.sum(z2 * z2, axis=0, keepdims=True)

    @pl.when(p == 2)
    def _():
        z2 = z2s[pl.ds(b * ROW_BLK, ROW_BLK), :]
        m = s2[...] / N
        v = ss2[...] / N - m * m
        hn = (z2 - m) * jax.lax.rsqrt(v + 1e-5) * g2[...] + be2[...]
        hn = jnp.maximum(hn, 0.0)
        half = hn.shape[1] // 2
        outL[...] = hn[:, :half]
        outR[...] = hn[:, half:]


def _head_layer_body(split_agg, nt,
                     hL, hR, aL, aR, eps, w1, b1, g1, be1, w2, b2, g2, be2,
                     batch, wc1, bc1, wc2, bc2, out,
                     z1s, z2s, s1, ss1, s2, ss2, psum, pcnt):
    p = pl.program_id(0)
    b = pl.program_id(1)

    @pl.when((p == 0) & (b == 0))
    def _():
        for ref in (s1, ss1, s2, ss2, psum, pcnt):
            ref[...] = jnp.zeros_like(ref)

    @pl.when(p == 0)
    def _():
        h = jnp.concatenate([hL[...], hR[...]], axis=1)
        if split_agg:
            a = jnp.concatenate([aL[...], aR[...]], axis=1)
        else:
            a = aL[...] + aR[...]
        y = (1.0 + eps[0, 0]) * h + a
        z1 = _dot(y, w1[...]) + b1[...]
        z1s[pl.ds(b * ROW_BLK, ROW_BLK), :] = z1
        s1[...] += jnp.sum(z1, axis=0, keepdims=True)
        ss1[...] += jnp.sum(z1 * z1, axis=0, keepdims=True)

    @pl.when(p == 1)
    def _():
        z1 = z1s[pl.ds(b * ROW_BLK, ROW_BLK), :]
        m = s1[...] / N
        v = ss1[...] / N - m * m
        a1 = (z1 - m) * jax.lax.rsqrt(v + 1e-5) * g1[...] + be1[...]
        a1 = jnp.maximum(a1, 0.0)
        z2 = _dot(a1, w2[...]) + b2[...]
        z2s[pl.ds(b * ROW_BLK, ROW_BLK), :] = z2
        s2[...] += jnp.sum(z2, axis=0, keepdims=True)
        ss2[...] += jnp.sum(z2 * z2, axis=0, keepdims=True)

    @pl.when(p == 2)
    def _():
        z2 = z2s[pl.ds(b * ROW_BLK, ROW_BLK), :]
        m = s2[...] / N
        v = ss2[...] / N - m * m
        hn = (z2 - m) * jax.lax.rsqrt(v + 1e-5) * g2[...] + be2[...]
        hn = jnp.maximum(hn, 0.0)
        oh = (batch[...] ==
              jax.lax.broadcasted_iota(jnp.int32, (ROW_BLK, B), 1))
        oh = oh.astype(_F32)
        psum[...] += jax.lax.dot_general(
            oh, hn, (((0,), (0,)), ((), ())), precision=_HI,
            preferred_element_type=_F32)
        pcnt[...] += jax.lax.dot_general(
            oh, jnp.ones((ROW_BLK, 128), _F32), (((0,), (0,)), ((), ())),
            precision=_HI, preferred_element_type=_F32)

    @pl.when((p == 3) & (b == 0))
    def _():
        pooled = psum[...] / jnp.maximum(pcnt[...][:, :1], 1.0)
        hid = jnp.maximum(_dot(pooled, wc1[...]) + bc1[...], 0.0)
        out[...] = _dot(hid, wc2[...]) + bc2[...]


@functools.cache
def _make_tc_layer(din, split_agg, nt=0):
    chin = din // 2
    cha = chin if split_agg else din
    hid2 = 512
    hid = 256
    cho = hid // 2
    fuse_head = nt > 0

    blk = lambda r, c: pl.BlockSpec(
        (r, c), lambda p, b: (jnp.where(p == 0, b, 0), 0))
    p2blk = lambda r, c: pl.BlockSpec(
        (r, c), lambda p, b: (jnp.where(p == 2, b, 0), 0))
    full = lambda r, c: pl.BlockSpec((r, c), lambda p, b: (0, 0))

    in_specs = [
        blk(ROW_BLK, chin), blk(ROW_BLK, chin),
        blk(ROW_BLK, cha), blk(ROW_BLK, cha),
        full(1, 1),
        full(din, hid2), full(1, hid2),
        full(1, hid2), full(1, hid2),
        full(hid2, hid), full(1, hid),
        full(1, hid), full(1, hid),
    ]
    scratch = [
        pltpu.VMEM((N, hid2), _F32),
        pltpu.VMEM((N, hid), _F32),
        pltpu.VMEM((1, hid2), _F32),
        pltpu.VMEM((1, hid2), _F32),
        pltpu.VMEM((1, hid), _F32),
        pltpu.VMEM((1, hid), _F32),
    ]
    if fuse_head:
        in_specs += [
            p2blk(ROW_BLK, 1),
            full(hid, hid // 2), full(1, hid // 2),
            full(hid // 2, nt), full(1, nt),
        ]
        scratch += [pltpu.VMEM((B, hid), _F32), pltpu.VMEM((B, 128), _F32)]
        return pl.pallas_call(
            functools.partial(_head_layer_body, split_agg, nt),
            grid=(4, NUM_BLKS),
            in_specs=in_specs,
            out_specs=[full(B, nt)],
            out_shape=[jax.ShapeDtypeStruct((B, nt), _F32)],
            scratch_shapes=scratch,
        )
    return pl.pallas_call(
        functools.partial(_layer_body, split_agg),
        grid=(3, NUM_BLKS),
        in_specs=in_specs,
        out_specs=[p2blk(ROW_BLK, cho), p2blk(ROW_BLK, cho)],
        out_shape=[jax.ShapeDtypeStruct((N, cho), _F32),
                   jax.ShapeDtypeStruct((N, cho), _F32)],
        scratch_shapes=scratch,
    )


def kernel(x, edge_index, batch, params):
    src = edge_index[0]
    dst = edge_index[1]
    npad = EPAD - E
    pad_src = (jnp.arange(npad, dtype=jnp.int32) * 37) % N
    pad_dst = N + (jnp.arange(npad, dtype=jnp.int32) % (NPAD - N))
    srcp = jnp.concatenate([src, pad_src])
    dstp = jnp.concatenate([dst, pad_dst])

    half = x.shape[1] // 2
    hL, hR = x[:, :half], x[:, half:]

    src_a = srcp.reshape(NUM_CORES * NUM_TILES, -1, K)
    dst_a = dstp.reshape(NUM_CORES * NUM_TILES, -1, K)
    src_b = srcp.reshape(NUM_TILES, -1, K)
    dst_b = dstp.reshape(NUM_TILES, -1, K)

    r1 = lambda a: a.reshape(1, -1)
    nt = params["Wc2"].shape[1]
    for i, lp in enumerate(params["layers"]):
        din = hL.shape[1] * 2
        if i == 0:
            aggA, aggB = _make_sc_agg_full(din)(x, src_a, dst_a)
        else:
            aggA, aggB = _make_sc_agg(hL.shape[1])(hL, hR, src_b, dst_b)
        args = [hL, hR, aggA, aggB, lp["eps"].reshape(1, 1),
                lp["W1"], r1(lp["b1"]), r1(lp["g1"]), r1(lp["be1"]),
                lp["W2"], r1(lp["b2"]), r1(lp["g2"]), r1(lp["be2"])]
        last = i == len(params["layers"]) - 1
        if last:
            args += [batch.reshape(N, 1),
                     params["Wc1"], r1(params["bc1"]),
                     params["Wc2"], r1(params["bc2"])]
            (logits,) = _make_tc_layer(din, i != 0, nt)(*args)
            return logits
        hL, hR = _make_tc_layer(din, i != 0)(*args)


# speedup vs baseline: 1.3529x; 1.0355x over previous
import functools

import jax
import jax.numpy as jnp
from jax import lax
from jax.experimental import pallas as pl
from jax.experimental.pallas import tpu as pltpu
from jax.experimental.pallas import tpu_sc as plsc

N = 10000
E = 320000
B = 256

NUM_TILES = 16
NUM_CORES = 2
K = 128
NPAD = 10240
EPAD = 327680
ROW_BLK = 2000
NUM_BLKS = N // ROW_BLK

_F32 = jnp.float32
_HI = jax.lax.Precision.DEFAULT


def _dot(a, b):
    return jax.lax.dot_general(a, b, (((1,), (0,)), ((), ())),
                               precision=_HI, preferred_element_type=_F32)


def _sc_zero_acc(acc, rows_v, sid, ch):
    rows_per_tile = NPAD // NUM_TILES

    @pl.loop(0, K)
    def _(r):
        @pl.loop(0, ch // 16)
        def _(c):
            rows_v[r, pl.ds(c * 16, 16)] = jnp.zeros((16,), _F32)

    @pl.loop(0, rows_per_tile // K)
    def _(j):
        pltpu.sync_copy(rows_v, acc.at[pl.ds(sid * rows_per_tile + j * K, K)])


def _sc_edge_pipeline(h_ref, srcE3, dstE3, w, src2d, dst2d, acc,
                      rows0, rows1, sem0, sem1, n_chunks, g_sz):
    @pl.loop(0, n_chunks // g_sz)
    def _(g):
        pltpu.async_copy(srcE3.at[w, pl.ds(g * g_sz, g_sz)], src2d, sem0)
        pltpu.async_copy(dstE3.at[w, pl.ds(g * g_sz, g_sz)], dst2d, sem1)
        pltpu.make_async_copy(srcE3.at[w, pl.ds(g * g_sz, g_sz)],
                              src2d, sem0).wait()
        pltpu.make_async_copy(dstE3.at[w, pl.ds(g * g_sz, g_sz)],
                              dst2d, sem1).wait()
        pltpu.async_copy(h_ref.at[src2d.at[0]], rows0, sem0)

        @pl.loop(0, g_sz // 2)
        def _(j):
            ci = 2 * j
            pltpu.make_async_copy(h_ref.at[src2d.at[0]], rows0, sem0).wait()
            pltpu.async_copy(h_ref.at[src2d.at[ci + 1]], rows1, sem1)
            pltpu.sync_copy(rows0, acc.at[dst2d.at[ci]], add=True)
            nxt = jnp.minimum(ci + 2, g_sz - 1)
            pltpu.make_async_copy(h_ref.at[src2d.at[0]], rows1, sem1).wait()
            pltpu.async_copy(h_ref.at[src2d.at[nxt]], rows0, sem0)
            pltpu.sync_copy(rows1, acc.at[dst2d.at[ci + 1]], add=True)

        pltpu.make_async_copy(h_ref.at[src2d.at[0]], rows0, sem0).wait()


def _sc_writeout(acc, out_ref, sid):
    out_rows = 624
    tail = N - NUM_TILES * out_rows
    pltpu.sync_copy(acc.at[pl.ds(sid * out_rows, out_rows)],
                    out_ref.at[pl.ds(sid * out_rows, out_rows)])

    @pl.when(sid == 0)
    def _():
        pltpu.sync_copy(acc.at[pl.ds(NUM_TILES * out_rows, tail)],
                        out_ref.at[pl.ds(NUM_TILES * out_rows, tail)])


def _sc_scratch(ch, g_sz):
    return [
        pltpu.VMEM_SHARED((NPAD, ch), _F32),
        pltpu.VMEM((g_sz, K), jnp.int32),
        pltpu.VMEM((g_sz, K), jnp.int32),
        pltpu.VMEM((K, ch), _F32),
        pltpu.VMEM((K, ch), _F32),
        pltpu.SemaphoreType.DMA,
        pltpu.SemaphoreType.DMA,
    ]


@functools.cache
def _make_sc_agg(ch):
    mesh = plsc.VectorSubcoreMesh(core_axis_name="c", subcore_axis_name="s")
    n_chunks = EPAD // (NUM_TILES * K)
    g_sz = 32

    @functools.partial(
        pl.kernel,
        mesh=mesh,
        out_type=[jax.ShapeDtypeStruct((N, ch), _F32),
                  jax.ShapeDtypeStruct((N, ch), _F32)],
        scratch_types=_sc_scratch(ch, g_sz),
    )
    def agg(hL, hR, srcE3, dstE3, aggL, aggR,
            acc, src2d, dst2d, rows0, rows1, sem0, sem1):
        cid = lax.axis_index("c")
        sid = lax.axis_index("s")
        _sc_zero_acc(acc, rows0, sid, ch)
        plsc.subcore_barrier()

        @pl.when(cid == 0)
        def _():
            _sc_edge_pipeline(hL, srcE3, dstE3, sid, src2d, dst2d, acc,
                              rows0, rows1, sem0, sem1, n_chunks, g_sz)

        @pl.when(cid == 1)
        def _():
            _sc_edge_pipeline(hR, srcE3, dstE3, sid, src2d, dst2d, acc,
                              rows0, rows1, sem0, sem1, n_chunks, g_sz)

        plsc.subcore_barrier()

        @pl.when(cid == 0)
        def _():
            _sc_writeout(acc, aggL, sid)

        @pl.when(cid == 1)
        def _():
            _sc_writeout(acc, aggR, sid)

    return agg


@functools.cache
def _make_sc_agg_full(ch):
    mesh = plsc.VectorSubcoreMesh(core_axis_name="c", subcore_axis_name="s")
    n_chunks = EPAD // (NUM_CORES * NUM_TILES * K)
    g_sz = 40

    @functools.partial(
        pl.kernel,
        mesh=mesh,
        out_type=[jax.ShapeDtypeStruct((N, ch), _F32),
                  jax.ShapeDtypeStruct((N, ch), _F32)],
        scratch_types=_sc_scratch(ch, g_sz),
    )
    def agg(h, srcE3, dstE3, agg_a, agg_b,
            acc, src2d, dst2d, rows0, rows1, sem0, sem1):
        cid = lax.axis_index("c")
        sid = lax.axis_index("s")
        _sc_zero_acc(acc, rows0, sid, ch)
        plsc.subcore_barrier()
        _sc_edge_pipeline(h, srcE3, dstE3, cid * NUM_TILES + sid,
                          src2d, dst2d, acc, rows0, rows1, sem0, sem1,
                          n_chunks, g_sz)
        plsc.subcore_barrier()

        @pl.when(cid == 0)
        def _():
            _sc_writeout(acc, agg_a, sid)

        @pl.when(cid == 1)
        def _():
            _sc_writeout(acc, agg_b, sid)

    return agg


def _layer_body(split_agg, hL, hR, aL, aR, eps, w1, b1, g1, be1, w2, b2, g2,
                be2, outL, outR, z1s, z2s, s1, ss1, s2, ss2):
    p = pl.program_id(0)
    b = pl.program_id(1)

    @pl.when((p == 0) & (b == 0))
    def _():
        s1[...] = jnp.zeros_like(s1)
        ss1[...] = jnp.zeros_like(ss1)
        s2[...] = jnp.zeros_like(s2)
        ss2[...] = jnp.zeros_like(ss2)

    @pl.when(p == 0)
    def _():
        h = jnp.concatenate([hL[...], hR[...]], axis=1)
        if split_agg:
            a = jnp.concatenate([aL[...], aR[...]], axis=1)
        else:
            a = aL[...] + aR[...]
        y = (1.0 + eps[0, 0]) * h + a
        z1 = _dot(y, w1[...]) + b1[...]
        z1s[pl.ds(b * ROW_BLK, ROW_BLK), :] = z1
        s1[...] += jnp.sum(z1, axis=0, keepdims=True)
        ss1[...] += jnp.sum(z1 * z1, axis=0, keepdims=True)

    @pl.when(p == 1)
    def _():
        z1 = z1s[pl.ds(b * ROW_BLK, ROW_BLK), :]
        m = s1[...] / N
        v = ss1[...] / N - m * m
        a1 = (z1 - m) * jax.lax.rsqrt(v + 1e-5) * g1[...] + be1[...]
        a1 = jnp.maximum(a1, 0.0)
        z2 = _dot(a1, w2[...]) + b2[...]
        z2s[pl.ds(b * ROW_BLK, ROW_BLK), :] = z2
        s2[...] += jnp.sum(z2, axis=0, keepdims=True)
        ss2[...] += jnp.sum(z2 * z2, axis=0, keepdims=True)

    @pl.when(p == 2)
    def _():
        z2 = z2s[pl.ds(b * ROW_BLK, ROW_BLK), :]
        m = s2[...] / N
        v = ss2[...] / N - m * m
        hn = (z2 - m) * jax.lax.rsqrt(v + 1e-5) * g2[...] + be2[...]
        hn = jnp.maximum(hn, 0.0)
        half = hn.shape[1] // 2
        outL[...] = hn[:, :half]
        outR[...] = hn[:, half:]


def _head_layer_body(split_agg, nt,
                     hL, hR, aL, aR, eps, w1, b1, g1, be1, w2, b2, g2, be2,
                     batch, wc1, bc1, wc2, bc2, out,
                     z1s, z2s, s1, ss1, s2, ss2, psum, pcnt):
    p = pl.program_id(0)
    b = pl.program_id(1)

    @pl.when((p == 0) & (b == 0))
    def _():
        for ref in (s1, ss1, s2, ss2, psum, pcnt):
            ref[...] = jnp.zeros_like(ref)

    @pl.when(p == 0)
    def _():
        h = jnp.concatenate([hL[...], hR[...]], axis=1)
        if split_agg:
            a = jnp.concatenate([aL[...], aR[...]], axis=1)
        else:
            a = aL[...] + aR[...]
        y = (1.0 + eps[0, 0]) * h + a
        z1 = _dot(y, w1[...]) + b1[...]
        z1s[pl.ds(b * ROW_BLK, ROW_BLK), :] = z1
        s1[...] += jnp.sum(z1, axis=0, keepdims=True)
        ss1[...] += jnp.sum(z1 * z1, axis=0, keepdims=True)

    @pl.when(p == 1)
    def _():
        z1 = z1s[pl.ds(b * ROW_BLK, ROW_BLK), :]
        m = s1[...] / N
        v = ss1[...] / N - m * m
        a1 = (z1 - m) * jax.lax.rsqrt(v + 1e-5) * g1[...] + be1[...]
        a1 = jnp.maximum(a1, 0.0)
        z2 = _dot(a1, w2[...]) + b2[...]
        z2s[pl.ds(b * ROW_BLK, ROW_BLK), :] = z2
        s2[...] += jnp.sum(z2, axis=0, keepdims=True)
        ss2[...] += jnp.sum(z2 * z2, axis=0, keepdims=True)

    @pl.when(p == 2)
    def _():
        z2 = z2s[pl.ds(b * ROW_BLK, ROW_BLK), :]
        m = s2[...] / N
        v = ss2[...] / N - m * m
        hn = (z2 - m) * jax.lax.rsqrt(v + 1e-5) * g2[...] + be2[...]
        hn = jnp.maximum(hn, 0.0)
        oh = (batch[...] ==
              jax.lax.broadcasted_iota(jnp.int32, (ROW_BLK, B), 1))
        oh = oh.astype(_F32)
        psum[...] += jax.lax.dot_general(
            oh, hn, (((0,), (0,)), ((), ())), precision=_HI,
            preferred_element_type=_F32)
        pcnt[...] += jax.lax.dot_general(
            oh, jnp.ones((ROW_BLK, 128), _F32), (((0,), (0,)), ((), ())),
            precision=_HI, preferred_element_type=_F32)

    @pl.when((p == 3) & (b == 0))
    def _():
        pooled = psum[...] / jnp.maximum(pcnt[...][:, :1], 1.0)
        hid = jnp.maximum(_dot(pooled, wc1[...]) + bc1[...], 0.0)
        out[...] = _dot(hid, wc2[...]) + bc2[...]


@functools.cache
def _make_tc_layer(din, split_agg, nt=0):
    chin = din // 2
    cha = chin if split_agg else din
    hid2 = 512
    hid = 256
    cho = hid // 2
    fuse_head = nt > 0

    blk = lambda r, c: pl.BlockSpec(
        (r, c), lambda p, b: (jnp.where(p == 0, b, 0), 0))
    p2blk = lambda r, c: pl.BlockSpec(
        (r, c), lambda p, b: (jnp.where(p == 2, b, 0), 0))
    full = lambda r, c: pl.BlockSpec((r, c), lambda p, b: (0, 0))

    in_specs = [
        blk(ROW_BLK, chin), blk(ROW_BLK, chin),
        blk(ROW_BLK, cha), blk(ROW_BLK, cha),
        full(1, 1),
        full(din, hid2), full(1, hid2),
        full(1, hid2), full(1, hid2),
        full(hid2, hid), full(1, hid),
        full(1, hid), full(1, hid),
    ]
    scratch = [
        pltpu.VMEM((N, hid2), _F32),
        pltpu.VMEM((N, hid), _F32),
        pltpu.VMEM((1, hid2), _F32),
        pltpu.VMEM((1, hid2), _F32),
        pltpu.VMEM((1, hid), _F32),
        pltpu.VMEM((1, hid), _F32),
    ]
    if fuse_head:
        in_specs += [
            p2blk(ROW_BLK, 1),
            full(hid, hid // 2), full(1, hid // 2),
            full(hid // 2, nt), full(1, nt),
        ]
        scratch += [pltpu.VMEM((B, hid), _F32), pltpu.VMEM((B, 128), _F32)]
        return pl.pallas_call(
            functools.partial(_head_layer_body, split_agg, nt),
            grid=(4, NUM_BLKS),
            in_specs=in_specs,
            out_specs=[full(B, nt)],
            out_shape=[jax.ShapeDtypeStruct((B, nt), _F32)],
            scratch_shapes=scratch,
        )
    return pl.pallas_call(
        functools.partial(_layer_body, split_agg),
        grid=(3, NUM_BLKS),
        in_specs=in_specs,
        out_specs=[p2blk(ROW_BLK, cho), p2blk(ROW_BLK, cho)],
        out_shape=[jax.ShapeDtypeStruct((N, cho), _F32),
                   jax.ShapeDtypeStruct((N, cho), _F32)],
        scratch_shapes=scratch,
    )


def kernel(x, edge_index, batch, params):
    src = edge_index[0]
    dst = edge_index[1]
    npad = EPAD - E
    pad_src = (jnp.arange(npad, dtype=jnp.int32) * 37) % N
    pad_dst = N + (jnp.arange(npad, dtype=jnp.int32) % (NPAD - N))
    srcp = jnp.concatenate([src, pad_src])
    dstp = jnp.concatenate([dst, pad_dst])

    half = x.shape[1] // 2
    hL, hR = x[:, :half], x[:, half:]

    src_a = srcp.reshape(NUM_CORES * NUM_TILES, -1, K)
    dst_a = dstp.reshape(NUM_CORES * NUM_TILES, -1, K)
    src_b = srcp.reshape(NUM_TILES, -1, K)
    dst_b = dstp.reshape(NUM_TILES, -1, K)

    r1 = lambda a: a.reshape(1, -1)
    nt = params["Wc2"].shape[1]
    for i, lp in enumerate(params["layers"]):
        din = hL.shape[1] * 2
        if i == 0:
            aggA, aggB = _make_sc_agg_full(din)(x, src_a, dst_a)
        else:
            aggA, aggB = _make_sc_agg(hL.shape[1])(hL, hR, src_b, dst_b)
        args = [hL, hR, aggA, aggB, lp["eps"].reshape(1, 1),
                lp["W1"], r1(lp["b1"]), r1(lp["g1"]), r1(lp["be1"]),
                lp["W2"], r1(lp["b2"]), r1(lp["g2"]), r1(lp["be2"])]
        last = i == len(params["layers"]) - 1
        if last:
            args += [batch.reshape(N, 1),
                     params["Wc1"], r1(params["bc1"]),
                     params["Wc2"], r1(params["bc2"])]
            (logits,) = _make_tc_layer(din, i != 0, nt)(*args)
            return logits
        hL, hR = _make_tc_layer(din, i != 0)(*args)


# speedup vs baseline: 1.3602x; 1.0054x over previous
import functools

import jax
import jax.numpy as jnp
from jax import lax
from jax.experimental import pallas as pl
from jax.experimental.pallas import tpu as pltpu
from jax.experimental.pallas import tpu_sc as plsc

N = 10000
E = 320000
B = 256

NUM_TILES = 16
NUM_CORES = 2
K = 128
NPAD = 10240
EPAD = 327680
ROW_BLK = 2000
NUM_BLKS = N // ROW_BLK

_F32 = jnp.float32
_HI = jax.lax.Precision.DEFAULT


def _dot(a, b):
    return jax.lax.dot_general(a, b, (((1,), (0,)), ((), ())),
                               precision=_HI, preferred_element_type=_F32)


def _sc_zero_acc(acc, rows_v, sid, ch):
    rows_per_tile = NPAD // NUM_TILES

    @pl.loop(0, K)
    def _(r):
        @pl.loop(0, ch // 16)
        def _(c):
            rows_v[r, pl.ds(c * 16, 16)] = jnp.zeros((16,), _F32)

    @pl.loop(0, rows_per_tile // K)
    def _(j):
        pltpu.sync_copy(rows_v, acc.at[pl.ds(sid * rows_per_tile + j * K, K)])


def _sc_edge_pipeline(h_ref, srcE3, dstE3, w, src2d, dst2d, acc,
                      rows0, rows1, sem0, sem1, n_chunks, g_sz):
    @pl.loop(0, n_chunks // g_sz)
    def _(g):
        pltpu.async_copy(srcE3.at[w, pl.ds(g * g_sz, g_sz)], src2d, sem0)
        pltpu.async_copy(dstE3.at[w, pl.ds(g * g_sz, g_sz)], dst2d, sem1)
        pltpu.make_async_copy(srcE3.at[w, pl.ds(g * g_sz, g_sz)],
                              src2d, sem0).wait()
        pltpu.make_async_copy(dstE3.at[w, pl.ds(g * g_sz, g_sz)],
                              dst2d, sem1).wait()
        pltpu.async_copy(h_ref.at[src2d.at[0]], rows0, sem0)

        @pl.loop(0, g_sz // 2)
        def _(j):
            ci = 2 * j
            pltpu.make_async_copy(h_ref.at[src2d.at[0]], rows0, sem0).wait()
            pltpu.async_copy(h_ref.at[src2d.at[ci + 1]], rows1, sem1)
            pltpu.sync_copy(rows0, acc.at[dst2d.at[ci]], add=True)
            nxt = jnp.minimum(ci + 2, g_sz - 1)
            pltpu.make_async_copy(h_ref.at[src2d.at[0]], rows1, sem1).wait()
            pltpu.async_copy(h_ref.at[src2d.at[nxt]], rows0, sem0)
            pltpu.sync_copy(rows1, acc.at[dst2d.at[ci + 1]], add=True)

        pltpu.make_async_copy(h_ref.at[src2d.at[0]], rows0, sem0).wait()


def _sc_writeout(acc, out_ref, sid):
    out_rows = 624
    tail = N - NUM_TILES * out_rows
    pltpu.sync_copy(acc.at[pl.ds(sid * out_rows, out_rows)],
                    out_ref.at[pl.ds(sid * out_rows, out_rows)])

    @pl.when(sid == 0)
    def _():
        pltpu.sync_copy(acc.at[pl.ds(NUM_TILES * out_rows, tail)],
                        out_ref.at[pl.ds(NUM_TILES * out_rows, tail)])


def _sc_scratch(ch, g_sz):
    return [
        pltpu.VMEM_SHARED((NPAD, ch), _F32),
        pltpu.VMEM((g_sz, K), jnp.int32),
        pltpu.VMEM((g_sz, K), jnp.int32),
        pltpu.VMEM((K, ch), _F32),
        pltpu.VMEM((K, ch), _F32),
        pltpu.SemaphoreType.DMA,
        pltpu.SemaphoreType.DMA,
    ]


@functools.cache
def _make_sc_agg(ch):
    mesh = plsc.VectorSubcoreMesh(core_axis_name="c", subcore_axis_name="s")
    n_chunks = EPAD // (NUM_TILES * K)
    g_sz = 40

    @functools.partial(
        pl.kernel,
        mesh=mesh,
        out_type=[jax.ShapeDtypeStruct((N, ch), _F32),
                  jax.ShapeDtypeStruct((N, ch), _F32)],
        scratch_types=_sc_scratch(ch, g_sz),
    )
    def agg(hL, hR, srcE3, dstE3, aggL, aggR,
            acc, src2d, dst2d, rows0, rows1, sem0, sem1):
        cid = lax.axis_index("c")
        sid = lax.axis_index("s")
        _sc_zero_acc(acc, rows0, sid, ch)
        plsc.subcore_barrier()

        @pl.when(cid == 0)
        def _():
            _sc_edge_pipeline(hL, srcE3, dstE3, sid, src2d, dst2d, acc,
                              rows0, rows1, sem0, sem1, n_chunks, g_sz)

        @pl.when(cid == 1)
        def _():
            _sc_edge_pipeline(hR, srcE3, dstE3, sid, src2d, dst2d, acc,
                              rows0, rows1, sem0, sem1, n_chunks, g_sz)

        plsc.subcore_barrier()

        @pl.when(cid == 0)
        def _():
            _sc_writeout(acc, aggL, sid)

        @pl.when(cid == 1)
        def _():
            _sc_writeout(acc, aggR, sid)

    return agg


@functools.cache
def _make_sc_agg_full(ch):
    mesh = plsc.VectorSubcoreMesh(core_axis_name="c", subcore_axis_name="s")
    n_chunks = EPAD // (NUM_CORES * NUM_TILES * K)
    g_sz = 40

    @functools.partial(
        pl.kernel,
        mesh=mesh,
        out_type=[jax.ShapeDtypeStruct((N, ch), _F32),
                  jax.ShapeDtypeStruct((N, ch), _F32)],
        scratch_types=_sc_scratch(ch, g_sz),
    )
    def agg(h, srcE3, dstE3, agg_a, agg_b,
            acc, src2d, dst2d, rows0, rows1, sem0, sem1):
        cid = lax.axis_index("c")
        sid = lax.axis_index("s")
        _sc_zero_acc(acc, rows0, sid, ch)
        plsc.subcore_barrier()
        _sc_edge_pipeline(h, srcE3, dstE3, cid * NUM_TILES + sid,
                          src2d, dst2d, acc, rows0, rows1, sem0, sem1,
                          n_chunks, g_sz)
        plsc.subcore_barrier()

        @pl.when(cid == 0)
        def _():
            _sc_writeout(acc, agg_a, sid)

        @pl.when(cid == 1)
        def _():
            _sc_writeout(acc, agg_b, sid)

    return agg


def _layer_body(split_agg, hL, hR, aL, aR, eps, w1, b1, g1, be1, w2, b2, g2,
                be2, outL, outR, z1s, z2s, s1, ss1, s2, ss2):
    p = pl.program_id(0)
    b = pl.program_id(1)

    @pl.when((p == 0) & (b == 0))
    def _():
        s1[...] = jnp.zeros_like(s1)
        ss1[...] = jnp.zeros_like(ss1)
        s2[...] = jnp.zeros_like(s2)
        ss2[...] = jnp.zeros_like(ss2)

    @pl.when(p == 0)
    def _():
        h = jnp.concatenate([hL[...], hR[...]], axis=1)
        if split_agg:
            a = jnp.concatenate([aL[...], aR[...]], axis=1)
        else:
            a = aL[...] + aR[...]
        y = (1.0 + eps[0, 0]) * h + a
        z1 = _dot(y, w1[...]) + b1[...]
        z1s[pl.ds(b * ROW_BLK, ROW_BLK), :] = z1
        s1[...] += jnp.sum(z1, axis=0, keepdims=True)
        ss1[...] += jnp.sum(z1 * z1, axis=0, keepdims=True)

    @pl.when(p == 1)
    def _():
        z1 = z1s[pl.ds(b * ROW_BLK, ROW_BLK), :]
        m = s1[...] / N
        v = ss1[...] / N - m * m
        a1 = (z1 - m) * jax.lax.rsqrt(v + 1e-5) * g1[...] + be1[...]
        a1 = jnp.maximum(a1, 0.0)
        z2 = _dot(a1, w2[...]) + b2[...]
        z2s[pl.ds(b * ROW_BLK, ROW_BLK), :] = z2
        s2[...] += jnp.sum(z2, axis=0, keepdims=True)
        ss2[...] += jnp.sum(z2 * z2, axis=0, keepdims=True)

    @pl.when(p == 2)
    def _():
        z2 = z2s[pl.ds(b * ROW_BLK, ROW_BLK), :]
        m = s2[...] / N
        v = ss2[...] / N - m * m
        hn = (z2 - m) * jax.lax.rsqrt(v + 1e-5) * g2[...] + be2[...]
        hn = jnp.maximum(hn, 0.0)
        half = hn.shape[1] // 2
        outL[...] = hn[:, :half]
        outR[...] = hn[:, half:]


def _head_layer_body(split_agg, nt,
                     hL, hR, aL, aR, eps, w1, b1, g1, be1, w2, b2, g2, be2,
                     batch, wc1, bc1, wc2, bc2, out,
                     z1s, z2s, s1, ss1, s2, ss2, psum, pcnt):
    p = pl.program_id(0)
    b = pl.program_id(1)

    @pl.when((p == 0) & (b == 0))
    def _():
        for ref in (s1, ss1, s2, ss2, psum, pcnt):
            ref[...] = jnp.zeros_like(ref)

    @pl.when(p == 0)
    def _():
        h = jnp.concatenate([hL[...], hR[...]], axis=1)
        if split_agg:
            a = jnp.concatenate([aL[...], aR[...]], axis=1)
        else:
            a = aL[...] + aR[...]
        y = (1.0 + eps[0, 0]) * h + a
        z1 = _dot(y, w1[...]) + b1[...]
        z1s[pl.ds(b * ROW_BLK, ROW_BLK), :] = z1
        s1[...] += jnp.sum(z1, axis=0, keepdims=True)
        ss1[...] += jnp.sum(z1 * z1, axis=0, keepdims=True)

    @pl.when(p == 1)
    def _():
        z1 = z1s[pl.ds(b * ROW_BLK, ROW_BLK), :]
        m = s1[...] / N
        v = ss1[...] / N - m * m
        a1 = (z1 - m) * jax.lax.rsqrt(v + 1e-5) * g1[...] + be1[...]
        a1 = jnp.maximum(a1, 0.0)
        z2 = _dot(a1, w2[...]) + b2[...]
        z2s[pl.ds(b * ROW_BLK, ROW_BLK), :] = z2
        s2[...] += jnp.sum(z2, axis=0, keepdims=True)
        ss2[...] += jnp.sum(z2 * z2, axis=0, keepdims=True)

    @pl.when(p == 2)
    def _():
        z2 = z2s[pl.ds(b * ROW_BLK, ROW_BLK), :]
        m = s2[...] / N
        v = ss2[...] / N - m * m
        hn = (z2 - m) * jax.lax.rsqrt(v + 1e-5) * g2[...] + be2[...]
        hn = jnp.maximum(hn, 0.0)
        oh = (batch[...] ==
              jax.lax.broadcasted_iota(jnp.int32, (ROW_BLK, B), 1))
        oh = oh.astype(_F32)
        psum[...] += jax.lax.dot_general(
            oh, hn, (((0,), (0,)), ((), ())), precision=_HI,
            preferred_element_type=_F32)
        pcnt[...] += jax.lax.dot_general(
            oh, jnp.ones((ROW_BLK, 128), _F32), (((0,), (0,)), ((), ())),
            precision=_HI, preferred_element_type=_F32)

    @pl.when((p == 3) & (b == 0))
    def _():
        pooled = psum[...] / jnp.maximum(pcnt[...][:, :1], 1.0)
        hid = jnp.maximum(_dot(pooled, wc1[...]) + bc1[...], 0.0)
        out[...] = _dot(hid, wc2[...]) + bc2[...]


@functools.cache
def _make_tc_layer(din, split_agg, nt=0):
    chin = din // 2
    cha = chin if split_agg else din
    hid2 = 512
    hid = 256
    cho = hid // 2
    fuse_head = nt > 0

    blk = lambda r, c: pl.BlockSpec(
        (r, c), lambda p, b: (jnp.where(p == 0, b, 0), 0))
    p2blk = lambda r, c: pl.BlockSpec(
        (r, c), lambda p, b: (jnp.where(p == 2, b, 0), 0))
    full = lambda r, c: pl.BlockSpec((r, c), lambda p, b: (0, 0))

    in_specs = [
        blk(ROW_BLK, chin), blk(ROW_BLK, chin),
        blk(ROW_BLK, cha), blk(ROW_BLK, cha),
        full(1, 1),
        full(din, hid2), full(1, hid2),
        full(1, hid2), full(1, hid2),
        full(hid2, hid), full(1, hid),
        full(1, hid), full(1, hid),
    ]
    scratch = [
        pltpu.VMEM((N, hid2), _F32),
        pltpu.VMEM((N, hid), _F32),
        pltpu.VMEM((1, hid2), _F32),
        pltpu.VMEM((1, hid2), _F32),
        pltpu.VMEM((1, hid), _F32),
        pltpu.VMEM((1, hid), _F32),
    ]
    if fuse_head:
        in_specs += [
            p2blk(ROW_BLK, 1),
            full(hid, hid // 2), full(1, hid // 2),
            full(hid // 2, nt), full(1, nt),
        ]
        scratch += [pltpu.VMEM((B, hid), _F32), pltpu.VMEM((B, 128), _F32)]
        return pl.pallas_call(
            functools.partial(_head_layer_body, split_agg, nt),
            grid=(4, NUM_BLKS),
            in_specs=in_specs,
            out_specs=[full(B, nt)],
            out_shape=[jax.ShapeDtypeStruct((B, nt), _F32)],
            scratch_shapes=scratch,
        )
    return pl.pallas_call(
        functools.partial(_layer_body, split_agg),
        grid=(3, NUM_BLKS),
        in_specs=in_specs,
        out_specs=[p2blk(ROW_BLK, cho), p2blk(ROW_BLK, cho)],
        out_shape=[jax.ShapeDtypeStruct((N, cho), _F32),
                   jax.ShapeDtypeStruct((N, cho), _F32)],
        scratch_shapes=scratch,
    )


def kernel(x, edge_index, batch, params):
    src = edge_index[0]
    dst = edge_index[1]
    npad = EPAD - E
    pad_src = (jnp.arange(npad, dtype=jnp.int32) * 37) % N
    pad_dst = N + (jnp.arange(npad, dtype=jnp.int32) % (NPAD - N))
    srcp = jnp.concatenate([src, pad_src])
    dstp = jnp.concatenate([dst, pad_dst])

    half = x.shape[1] // 2
    hL, hR = x[:, :half], x[:, half:]

    src_a = srcp.reshape(NUM_CORES * NUM_TILES, -1, K)
    dst_a = dstp.reshape(NUM_CORES * NUM_TILES, -1, K)
    src_b = srcp.reshape(NUM_TILES, -1, K)
    dst_b = dstp.reshape(NUM_TILES, -1, K)

    r1 = lambda a: a.reshape(1, -1)
    nt = params["Wc2"].shape[1]
    for i, lp in enumerate(params["layers"]):
        din = hL.shape[1] * 2
        if i == 0:
            aggA, aggB = _make_sc_agg_full(din)(x, src_a, dst_a)
        else:
            aggA, aggB = _make_sc_agg(hL.shape[1])(hL, hR, src_b, dst_b)
        args = [hL, hR, aggA, aggB, lp["eps"].reshape(1, 1),
                lp["W1"], r1(lp["b1"]), r1(lp["g1"]), r1(lp["be1"]),
                lp["W2"], r1(lp["b2"]), r1(lp["g2"]), r1(lp["be2"])]
        last = i == len(params["layers"]) - 1
        if last:
            args += [batch.reshape(N, 1),
                     params["Wc1"], r1(params["bc1"]),
                     params["Wc2"], r1(params["bc2"])]
            (logits,) = _make_tc_layer(din, i != 0, nt)(*args)
            return logits
        hL, hR = _make_tc_layer(din, i != 0)(*args)
